# baseline (speedup 1.0000x reference)
import functools

import jax
import jax.numpy as jnp
from jax import lax
from jax.experimental import pallas as pl
from jax.experimental.pallas import tpu as pltpu
from jax.experimental.pallas import tpu_sc as plsc

_REG = 0.01
_EMB = 32
_NW = 32
_IDX = 128
_GPL = 128 // _EMB
_BLK = 2048


def _gather(B):
    BPW = B // _NW
    G = BPW // _IDX
    mesh = plsc.VectorSubcoreMesh(core_axis_name="c", subcore_axis_name="s")

    @functools.partial(
        pl.kernel,
        mesh=mesh,
        out_type=[jax.ShapeDtypeStruct((B, 128), jnp.float32)] * 3,
        scratch_types=[
            pltpu.VMEM((3, G, _IDX), jnp.int32),
            pltpu.VMEM((BPW, 128), jnp.float32),
            pltpu.SemaphoreType.DMA,
        ],
    )
    def k(gidx_h, p_h, q_h, r_h, gp_h, gq_h, gr_h, gidx, buf, sem):
        wid = lax.axis_index("c") * 16 + lax.axis_index("s")
        base = wid * BPW
        pltpu.sync_copy(gidx_h.at[wid], gidx)
        for t, (tab, out) in enumerate(((p_h, gp_h), (q_h, gq_h),
                                        (r_h, gr_h))):
            cps = [
                pltpu.async_copy(tab.at[gidx.at[t, g]],
                                 buf.at[pl.ds(g * _IDX, _IDX), :], sem)
                for g in range(G)
            ]
            for c in cps:
                c.wait()
            pltpu.sync_copy(buf, out.at[pl.ds(base, BPW)])

    return k


def _dense_body(nblk, gp_ref, gq_ref, gr_ref, op_ref, oq_ref, or_ref,
                w_ref, inf_ref, reg_ref, acc):
    i = pl.program_id(0)

    @pl.when(i == 0)
    def _():
        acc[...] = jnp.zeros_like(acc)

    w = w_ref[...]
    c = jnp.sqrt(jnp.sum(w * w))
    wn = w / jnp.maximum(c, 1.0)

    inf = jnp.zeros((gp_ref.shape[0], 1), jnp.float32)
    sq = []
    for t, (g_ref, o_ref) in enumerate(((gp_ref, op_ref), (gq_ref, oq_ref),
                                        (gr_ref, or_ref))):
        grp = g_ref[...]
        off = o_ref[...]
        sel = jnp.zeros((grp.shape[0], _EMB), jnp.float32)
        for kq in range(_GPL):
            sel = sel + jnp.where(off == float(kq),
                                  grp[:, kq * _EMB:(kq + 1) * _EMB], 0.0)
        wt = wn[0:1, t * _EMB:(t + 1) * _EMB]
        inf = inf + jnp.sum(sel * wt, axis=1, keepdims=True)
        sq.append(jnp.sum(sel * sel))
    inf_ref[...] = inf
    acc[...] = acc[...] + jnp.reshape(
        jnp.stack([sq[0], sq[1], sq[2], jnp.float32(0.0)]), (1, 4))

    @pl.when(i == nblk - 1)
    def _():
        s = acc[...]
        reg_ref[...] = jnp.reshape(_REG * jnp.sum(jnp.sqrt(s)), (1, 1))


def kernel(ps, qs, rs, P, Q, R, W):
    B = ps.shape[0]
    idx = jnp.stack([ps, qs, rs]).astype(jnp.int32)
    gidx = (idx // _GPL).reshape(3, _NW, -1, _IDX).transpose(1, 0, 2, 3)
    offs = (idx % _GPL).astype(jnp.float32)
    gp, gq, gr = _gather(B)(gidx, P.reshape(-1, 128), Q.reshape(-1, 128),
                            R.reshape(-1, 128))
    nblk = B // _BLK
    row_spec = pl.BlockSpec((_BLK, 128), lambda i: (i, 0))
    off_spec = pl.BlockSpec((_BLK, 1), lambda i: (i, 0))
    inf, reg = pl.pallas_call(
        functools.partial(_dense_body, nblk),
        grid=(nblk,),
        in_specs=[row_spec, row_spec, row_spec,
                  off_spec, off_spec, off_spec,
                  pl.BlockSpec((1, 96), lambda i: (0, 0))],
        out_specs=[pl.BlockSpec((_BLK, 1), lambda i: (i, 0)),
                   pl.BlockSpec((1, 1), lambda i: (0, 0))],
        out_shape=[
            jax.ShapeDtypeStruct((B, 1), jnp.float32),
            jax.ShapeDtypeStruct((1, 1), jnp.float32),
        ],
        scratch_shapes=[pltpu.VMEM((1, 4), jnp.float32)],
    )(gp, gq, gr, offs[0].reshape(B, 1), offs[1].reshape(B, 1),
      offs[2].reshape(B, 1), W.astype(jnp.float32))
    return inf, reg[0, 0]


# speedup vs baseline: 8.7191x; 8.7191x over previous
import functools

import jax
import jax.numpy as jnp
from jax import lax
from jax.experimental import pallas as pl
from jax.experimental.pallas import tpu as pltpu
from jax.experimental.pallas import tpu_sc as plsc

_REG = 0.01
_EMB = 32
_NW = 32
_IDX = 128
_BK = 32768

def _dense_body(pt_ref, qt_ref, rt_ref, w_ref,
                pwp_ref, pwq_ref, pwr_ref, spp_ref, spq_ref, spr_ref):
    ones = jnp.ones((1, _EMB), jnp.float32)
    for t, (t_ref, pw_ref, sp_ref) in enumerate(
            ((pt_ref, pwp_ref, spp_ref), (qt_ref, pwq_ref, spq_ref),
             (rt_ref, pwr_ref, spr_ref))):
        x = t_ref[...]
        wt = w_ref[0:1, t * _EMB:(t + 1) * _EMB]
        pw = jax.lax.dot_general(
            wt, x, (((1,), (0,)), ((), ())),
            preferred_element_type=jnp.float32)
        sp = jax.lax.dot_general(
            ones, x * x, (((1,), (0,)), ((), ())),
            preferred_element_type=jnp.float32)
        pw_ref[...] = pw.reshape(pw_ref.shape)
        sp_ref[...] = sp.reshape(sp_ref.shape)


def _combine(B):
    BPW = B // _NW
    G = BPW // _IDX
    mesh = plsc.VectorSubcoreMesh(core_axis_name="c", subcore_axis_name="s")

    @functools.partial(
        pl.kernel,
        mesh=mesh,
        out_type=[
            jax.ShapeDtypeStruct((B,), jnp.float32),
            jax.ShapeDtypeStruct((_NW, 48), jnp.float32),
        ],
        scratch_types=[
            pltpu.VMEM((3, G, _IDX), jnp.int32),
            pltpu.VMEM((BPW,), jnp.float32),
            pltpu.VMEM((BPW,), jnp.float32),
            pltpu.VMEM((BPW,), jnp.float32),
            pltpu.VMEM((48,), jnp.float32),
            pltpu.SemaphoreType.DMA,
        ],
    )
    def k(gidx_h, pwp_h, pwq_h, pwr_h, spp_h, spq_h, spr_h,
          inf_h, parts_h, gidx, pwv, spv, infv, sqv, sem):
        wid = lax.axis_index("c") * 16 + lax.axis_index("s")
        base = wid * BPW
        pltpu.sync_copy(gidx_h.at[wid], gidx)

        for t, (pw_h, sp_h) in enumerate(((pwp_h, spp_h), (pwq_h, spq_h),
                                          (pwr_h, spr_h))):
            cps = []
            for g in range(G):
                cps.append(pltpu.async_copy(
                    pw_h.at[gidx.at[t, g]],
                    pwv.at[pl.ds(g * _IDX, _IDX)], sem))
                cps.append(pltpu.async_copy(
                    sp_h.at[gidx.at[t, g]],
                    spv.at[pl.ds(g * _IDX, _IDX)], sem))
            for c in cps:
                c.wait()

            def chunk(v, sq, t=t):
                sl = pl.ds(v * 16, 16)
                if t == 0:
                    infv[sl] = pwv[sl]
                else:
                    infv[sl] = infv[sl] + pwv[sl]
                return sq + spv[sl]

            sq = lax.fori_loop(0, BPW // 16, chunk,
                               jnp.zeros((16,), jnp.float32))
            sqv[pl.ds(16 * t, 16)] = sq

        pltpu.sync_copy(infv, inf_h.at[pl.ds(base, BPW)])
        pltpu.sync_copy(sqv, parts_h.at[wid])

    return k


def kernel(ps, qs, rs, P, Q, R, W):
    B = ps.shape[0]
    wf = W.reshape(-1).astype(jnp.float32)
    wc = wf / jnp.maximum(jnp.sqrt(jnp.sum(wf * wf)), 1.0)
    nblk = pl.cdiv(P.shape[0], _BK)
    tab_spec = pl.BlockSpec((_EMB, _BK), lambda i: (0, i))
    vec_spec = pl.BlockSpec((_BK,), lambda i: (i,))
    pwp, pwq, pwr, spp, spq, spr = pl.pallas_call(
        _dense_body,
        grid=(nblk,),
        in_specs=[tab_spec, tab_spec, tab_spec,
                  pl.BlockSpec((1, 3 * _EMB), lambda i: (0, 0))],
        out_specs=[vec_spec] * 6,
        out_shape=[jax.ShapeDtypeStruct((P.shape[0],), jnp.float32)] * 6,
    )(P.T, Q.T, R.T, wc.reshape(1, 3 * _EMB))

    idx = jnp.stack([ps, qs, rs]).astype(jnp.int32)
    gidx = idx.reshape(3, _NW, -1, _IDX).transpose(1, 0, 2, 3)
    inf, parts = _combine(B)(gidx, pwp, pwq, pwr, spp, spq, spr)
    s = parts.reshape(_NW, 3, 16).sum(axis=(0, 2))
    regs = _REG * (jnp.sqrt(s[0]) + jnp.sqrt(s[1]) + jnp.sqrt(s[2]))
    return inf.reshape(B, 1), regs
